# baseline (speedup 1.0000x reference)
import functools

import jax
import jax.numpy as jnp
from jax.experimental import pallas as pl
from jax.experimental.pallas import tpu as pltpu

_B = 4096
_BLK = 256


_LOG2E = 1.4426950408889634


def _tree_reduce(parts, op):
    while len(parts) > 1:
        nxt = [op(parts[j], parts[j + 1]) for j in range(0, len(parts) - 1, 2)]
        if len(parts) % 2:
            nxt.append(parts[-1])
        parts = nxt
    return parts[0]


def _loss_block_kernel(logits_ref, bias_ref, out_ref):
    i = pl.program_id(0)
    bias = bias_ref[0]
    bias2 = bias * _LOG2E
    rows8 = _BLK // 8
    log2_acc = jnp.zeros((8, 128), jnp.float32)
    relu_acc = jnp.zeros((8, 128), jnp.float32)
    n = logits_ref.shape[1] // 128
    for k in range(n):
        accs_p = [None] * 8
        accs_r = [None] * 8
        for r in range(rows8):
            y = logits_ref[r * 8:(r + 1) * 8, k * 128:(k + 1) * 128] * _LOG2E + bias2
            neg_abs = jax.lax.bitcast_convert_type(
                jax.lax.bitcast_convert_type(y, jnp.uint32) | jnp.uint32(0x80000000),
                jnp.float32)
            t = 1.0 + jnp.exp2(neg_abs)
            rr = jnp.maximum(y, 0.0)
            j = r % 8
            accs_p[j] = t if accs_p[j] is None else accs_p[j] * t
            accs_r[j] = rr if accs_r[j] is None else accs_r[j] + rr
        p = _tree_reduce(accs_p, jnp.multiply)
        log2_acc = log2_acc + jnp.log2(p)
        relu_acc = relu_acc + _tree_reduce(accs_r, jnp.add)
    s = (jnp.sum(log2_acc) + jnp.sum(relu_acc)) * 0.6931471805599453
    rowi = jax.lax.broadcasted_iota(jnp.int32, (8, 128), 0)
    coli = jax.lax.broadcasted_iota(jnp.int32, (8, 128), 1)
    dacc = jnp.zeros((8, 128), jnp.float32)
    for m in range(rows8):
        c0 = (8 * m) // 128 * 128
        tile = logits_ref[8 * m:8 * m + 8, pl.ds(i * _BLK + c0, 128)]
        dacc = dacc + jnp.where(coli == rowi + (8 * m - c0), tile, 0.0)
    diag_sum = jnp.sum(dacc) + _BLK * bias
    out_ref[0, 0, 0] = s - diag_sum


@jax.jit
def kernel(logits, text_emb, image_emb, logit_bias):
    B = logits.shape[0]
    bias = jnp.reshape(logit_bias, (1,)).astype(jnp.float32)
    partials = pl.pallas_call(
        _loss_block_kernel,
        grid=(B // _BLK,),
        in_specs=[
            pl.BlockSpec((_BLK, B), lambda i: (i, 0)),
            pl.BlockSpec(memory_space=pltpu.SMEM),
        ],
        out_specs=pl.BlockSpec((1, 1, 1), lambda i: (i, 0, 0), memory_space=pltpu.SMEM),
        out_shape=jax.ShapeDtypeStruct((B // _BLK, 1, 1), jnp.float32),
        compiler_params=pltpu.CompilerParams(
            dimension_semantics=("parallel",),
        ),
    )(logits, bias)
    return jnp.sum(partials) / (B * B)


# speedup vs baseline: 1.1307x; 1.1307x over previous
import functools

import jax
import jax.numpy as jnp
from jax.experimental import pallas as pl
from jax.experimental.pallas import tpu as pltpu

_B = 4096
_BLK = 1024


_LOG2E = 1.4426950408889634


def _tree_reduce(parts, op):
    while len(parts) > 1:
        nxt = [op(parts[j], parts[j + 1]) for j in range(0, len(parts) - 1, 2)]
        if len(parts) % 2:
            nxt.append(parts[-1])
        parts = nxt
    return parts[0]


def _loss_block_kernel(logits_ref, bias_ref, out_ref):
    i = pl.program_id(0)
    bias = bias_ref[0]
    bias2 = bias * _LOG2E
    rows8 = _BLK // 8
    log2_acc = jnp.zeros((8, 128), jnp.float32)
    relu_acc = jnp.zeros((8, 128), jnp.float32)
    n = logits_ref.shape[1] // 128
    for k in range(n):
        accs_p = [None] * 8
        accs_r = [None] * 8
        for r in range(rows8):
            y = logits_ref[r * 8:(r + 1) * 8, k * 128:(k + 1) * 128] * _LOG2E + bias2
            neg_abs = jax.lax.bitcast_convert_type(
                jax.lax.bitcast_convert_type(y, jnp.uint32) | jnp.uint32(0x80000000),
                jnp.float32)
            t = 1.0 + jnp.exp2(neg_abs)
            rr = jnp.maximum(y, 0.0)
            j = r % 8
            accs_p[j] = t if accs_p[j] is None else accs_p[j] * t
            accs_r[j] = rr if accs_r[j] is None else accs_r[j] + rr
            if r % 64 == 63:
                p = _tree_reduce(accs_p, jnp.multiply)
                log2_acc = log2_acc + jnp.log2(p)
                accs_p = [None] * 8
        if accs_p[0] is not None:
            p = _tree_reduce([a for a in accs_p if a is not None], jnp.multiply)
            log2_acc = log2_acc + jnp.log2(p)
        relu_acc = relu_acc + _tree_reduce(accs_r, jnp.add)
    s = (jnp.sum(log2_acc) + jnp.sum(relu_acc)) * 0.6931471805599453
    rowi = jax.lax.broadcasted_iota(jnp.int32, (8, 128), 0)
    coli = jax.lax.broadcasted_iota(jnp.int32, (8, 128), 1)
    dacc = jnp.zeros((8, 128), jnp.float32)
    for m in range(rows8):
        c0 = (8 * m) // 128 * 128
        tile = logits_ref[8 * m:8 * m + 8, pl.ds(i * _BLK + c0, 128)]
        dacc = dacc + jnp.where(coli == rowi + (8 * m - c0), tile, 0.0)
    diag_sum = jnp.sum(dacc) + _BLK * bias
    out_ref[0, 0, 0] = s - diag_sum


@jax.jit
def kernel(logits, text_emb, image_emb, logit_bias):
    B = logits.shape[0]
    bias = jnp.reshape(logit_bias, (1,)).astype(jnp.float32)
    partials = pl.pallas_call(
        _loss_block_kernel,
        grid=(B // _BLK,),
        in_specs=[
            pl.BlockSpec((_BLK, B), lambda i: (i, 0)),
            pl.BlockSpec(memory_space=pltpu.SMEM),
        ],
        out_specs=pl.BlockSpec((1, 1, 1), lambda i: (i, 0, 0), memory_space=pltpu.SMEM),
        out_shape=jax.ShapeDtypeStruct((B // _BLK, 1, 1), jnp.float32),
        compiler_params=pltpu.CompilerParams(
            dimension_semantics=("parallel",),
        ),
    )(logits, bias)
    return jnp.sum(partials) / (B * B)
